# Initial kernel scaffold; baseline (speedup 1.0000x reference)
import jax
import jax.numpy as jnp
from jax.experimental import pallas as pl


def kernel(ids, table):
    raise NotImplementedError("write your pallas kernel here")


# speedup vs baseline: 1.2358x; 1.2358x over previous
import functools

import jax
import jax.numpy as jnp
from jax import lax
from jax.experimental import pallas as pl
from jax.experimental.pallas import tpu as pltpu
from jax.experimental.pallas import tpu_sc as plsc

B = 16384
D = 768
NC = 2
NS = 16
NW = NC * NS
BPW = B // NW
CHUNK = 64
NCHUNK = BPW // CHUNK


def _make_gather(V):
    mesh = plsc.VectorSubcoreMesh(core_axis_name="c", subcore_axis_name="s")

    @functools.partial(
        pl.kernel,
        mesh=mesh,
        out_type=jax.ShapeDtypeStruct((B, D), jnp.float32),
        scratch_types=[
            pltpu.VMEM((CHUNK,), jnp.int32),
            pltpu.VMEM((CHUNK, D), jnp.float32),
            pltpu.SemaphoreType.DMA,
        ],
    )
    def gather_kernel(table_hbm, idx_hbm, out_hbm, idx_v, rows_v, sem):
        wid = lax.axis_index("s") * NC + lax.axis_index("c")
        base = wid * BPW
        for c in range(NCHUNK):
            row0 = base + c * CHUNK
            pltpu.sync_copy(idx_hbm.at[pl.ds(row0, CHUNK)], idx_v)
            pltpu.async_copy(table_hbm.at[idx_v], rows_v, sem).wait()
            pltpu.sync_copy(rows_v, out_hbm.at[pl.ds(row0, CHUNK)])

    return gather_kernel


def kernel(ids, table):
    ids = ids.astype(jnp.int32)
    out = _make_gather(table.shape[0])(table, ids)
    return out[:, None, :]


# speedup vs baseline: 1.2889x; 1.0430x over previous
import functools

import jax
import jax.numpy as jnp
from jax import lax
from jax.experimental import pallas as pl
from jax.experimental.pallas import tpu as pltpu
from jax.experimental.pallas import tpu_sc as plsc

B = 16384
D = 768
NC = 2
NS = 16
NW = NC * NS
BPW = B // NW
CHUNK = 64
NCHUNK = BPW // CHUNK


def _make_gather(V):
    mesh = plsc.VectorSubcoreMesh(core_axis_name="c", subcore_axis_name="s")

    @functools.partial(
        pl.kernel,
        mesh=mesh,
        out_type=jax.ShapeDtypeStruct((B, D), jnp.float32),
        scratch_types=[
            pltpu.VMEM((NCHUNK, CHUNK), jnp.int32),
            pltpu.VMEM((2, CHUNK, D), jnp.float32),
            pltpu.SemaphoreType.DMA,
            pltpu.SemaphoreType.DMA,
            pltpu.SemaphoreType.DMA,
            pltpu.SemaphoreType.DMA,
        ],
    )
    def gather_kernel(table_hbm, idx_hbm, out_hbm, idx_v, rows_v,
                      gsem0, gsem1, osem0, osem1):
        wid = lax.axis_index("s") * NC + lax.axis_index("c")
        base = wid * BPW
        gsem = (gsem0, gsem1)
        osem = (osem0, osem1)
        for c in range(NCHUNK):
            pltpu.sync_copy(idx_hbm.at[pl.ds(base + c * CHUNK, CHUNK)],
                            idx_v.at[c])

        gh = [None, None]
        oh = [None, None]
        gh[0] = pltpu.async_copy(table_hbm.at[idx_v.at[0]], rows_v.at[0],
                                 gsem[0])
        for c in range(NCHUNK):
            buf = c % 2
            nb = 1 - buf
            if c + 1 < NCHUNK:
                if oh[nb] is not None:
                    oh[nb].wait()
                gh[nb] = pltpu.async_copy(table_hbm.at[idx_v.at[c + 1]],
                                          rows_v.at[nb], gsem[nb])
            gh[buf].wait()
            oh[buf] = pltpu.async_copy(
                rows_v.at[buf], out_hbm.at[pl.ds(base + c * CHUNK, CHUNK)],
                osem[buf])
        oh[0].wait()
        oh[1].wait()

    return gather_kernel


def kernel(ids, table):
    ids = ids.astype(jnp.int32)
    out = _make_gather(table.shape[0])(table, ids)
    return out[:, None, :]


# speedup vs baseline: 2.1113x; 1.6380x over previous
import functools

import jax
import jax.numpy as jnp
from jax import lax
from jax.experimental import pallas as pl
from jax.experimental.pallas import tpu as pltpu
from jax.experimental.pallas import tpu_sc as plsc

B = 16384
D = 768
NC = 2
NS = 16
NW = NC * NS
BPW = B // NW
CHUNK = 64
NCHUNK = BPW // CHUNK


def _make_gather(V):
    mesh = plsc.VectorSubcoreMesh(core_axis_name="c", subcore_axis_name="s")

    @functools.partial(
        pl.kernel,
        mesh=mesh,
        out_type=jax.ShapeDtypeStruct((B, 1, D), jnp.float32),
        scratch_types=[
            pltpu.VMEM((NCHUNK, CHUNK), jnp.int32),
            pltpu.VMEM((2, CHUNK, D), jnp.float32),
            pltpu.SemaphoreType.DMA,
            pltpu.SemaphoreType.DMA,
            pltpu.SemaphoreType.DMA,
            pltpu.SemaphoreType.DMA,
        ],
    )
    def gather_kernel(table_hbm, idx_hbm, out_hbm, idx_v, rows_v,
                      gsem0, gsem1, osem0, osem1):
        wid = lax.axis_index("s") * NC + lax.axis_index("c")
        base = wid * BPW
        gsem = (gsem0, gsem1)
        osem = (osem0, osem1)
        for c in range(NCHUNK):
            pltpu.sync_copy(idx_hbm.at[pl.ds(base + c * CHUNK, CHUNK)],
                            idx_v.at[c])

        gh = [None, None]
        oh = [None, None]
        gh[0] = pltpu.async_copy(table_hbm.at[idx_v.at[0]], rows_v.at[0],
                                 gsem[0])
        for c in range(NCHUNK):
            buf = c % 2
            nb = 1 - buf
            if c + 1 < NCHUNK:
                if oh[nb] is not None:
                    oh[nb].wait()
                gh[nb] = pltpu.async_copy(table_hbm.at[idx_v.at[c + 1]],
                                          rows_v.at[nb], gsem[nb])
            gh[buf].wait()
            oh[buf] = pltpu.async_copy(
                rows_v.at[buf],
                out_hbm.at[pl.ds(base + c * CHUNK, CHUNK), 0],
                osem[buf])
        oh[0].wait()
        oh[1].wait()

    return gather_kernel


def kernel(ids, table):
    ids = ids.astype(jnp.int32)
    return _make_gather(table.shape[0])(table, ids)
